# baseline (speedup 1.0000x reference)
import functools

import jax
import jax.numpy as jnp
from jax import lax
from jax.experimental import pallas as pl
from jax.experimental.pallas import tpu as pltpu
from jax.experimental.pallas import tpu_sc as plsc

B = 16384
S = 77
D = 64
EOS = 49407
NC = 2
NS = 16
NW = NC * NS
XCOLS = 2 * S * D
NPAIR = B // 2
ROWS_PER_W = NPAIR // NW
NCHUNK = ROWS_PER_W
NBUF = 4
LANES = 16


def _emb_body(ids_hbm, table_hbm, pos_hbm, out_hbm, pos_v,
              idx0, idx1, idx2, idx3, ge0, ge1, ge2, ge3,
              go0, go1, go2, go3, sb0, sb1, sb2, sb3,
              si0, si1, si2, si3, sg0, sg1, sg2, sg3, ss0, ss1, ss2, ss3):
    idx = [idx0, idx1, idx2, idx3]
    ge = [ge0, ge1, ge2, ge3]
    go = [go0, go1, go2, go3]
    sb = [sb0, sb1, sb2, sb3]
    si = [si0, si1, si2, si3]
    sg = [sg0, sg1, sg2, sg3]
    ss = [ss0, ss1, ss2, ss3]

    wid = lax.axis_index("s") * NC + lax.axis_index("c")
    base_xrow = wid * ROWS_PER_W

    pltpu.sync_copy(pos_hbm, pos_v)

    def start_idx(b, g):
        r = base_xrow + g
        pltpu.async_copy(ids_hbm.at[pl.ds(2 * r, 2)], idx[b], si[b])

    def wait_idx(b, g):
        r = base_xrow + g
        pltpu.make_async_copy(ids_hbm.at[pl.ds(2 * r, 2)], idx[b],
                              si[b]).wait()

    def start_gathers(b):
        pltpu.async_copy(table_hbm.at[idx[b].at[0]], ge[b], sg[b])
        pltpu.async_copy(table_hbm.at[idx[b].at[1]], go[b], sg[b])

    def wait_gathers(b):
        pltpu.make_async_copy(table_hbm.at[idx[b].at[0]], ge[b], sg[b]).wait()
        pltpu.make_async_copy(table_hbm.at[idx[b].at[1]], go[b], sg[b]).wait()

    def add_pos(b):
        def s_body(s, carry):
            col = s * D
            for j in range(D // LANES):
                p = pos_v[s, pl.ds(j * LANES, LANES)]
                sb[b][pl.ds(col + j * LANES, LANES)] = (
                    ge[b][s, pl.ds(j * LANES, LANES)] + p)
                sb[b][pl.ds(S * D + col + j * LANES, LANES)] = (
                    go[b][s, pl.ds(j * LANES, LANES)] + p)
            return carry

        lax.fori_loop(0, S, s_body, 0)

    def start_store(b, g):
        r = base_xrow + g
        pltpu.async_copy(sb[b], out_hbm.at[pl.ds(r * XCOLS, XCOLS)], ss[b])

    def wait_store(b, g):
        r = base_xrow + g
        pltpu.make_async_copy(sb[b], out_hbm.at[pl.ds(r * XCOLS, XCOLS)],
                              ss[b]).wait()

    for h in range(3):
        start_idx(h, h)
    for h in range(2):
        wait_idx(h, h)
        start_gathers(h)

    def outer_body(i, carry):
        for bb in range(NBUF):
            g = i * NBUF + bb
            b = bb
            wait_gathers(b)
            add_pos(b)
            start_store(b, g)
            bn = (bb + 2) % NBUF
            bi = (bb + 3) % NBUF

            @pl.when(g + 2 < NCHUNK)
            def _():
                @pl.when(g >= 2)
                def _():
                    wait_store(bn, g - 2)

                wait_idx(bn, g + 2)
                start_gathers(bn)

            @pl.when(g + 3 < NCHUNK)
            def _():
                start_idx(bi, g + 3)

        return carry

    lax.fori_loop(0, NCHUNK // NBUF, outer_body, 0)

    for k in range(NBUF):
        g = NCHUNK - NBUF + k
        wait_store(g % NBUF, g)


_scr_idx = [pltpu.VMEM((2, S), jnp.int32) for _ in range(NBUF)]
_scr_ge = [pltpu.VMEM((S, D), jnp.float32) for _ in range(NBUF)]
_scr_go = [pltpu.VMEM((S, D), jnp.float32) for _ in range(NBUF)]
_scr_sb = [pltpu.VMEM((XCOLS,), jnp.float32) for _ in range(NBUF)]
_scr_sem = [pltpu.SemaphoreType.DMA for _ in range(3 * NBUF)]

_emb = functools.partial(
    pl.kernel,
    out_type=jax.ShapeDtypeStruct((B * S * D,), jnp.float32),
    mesh=plsc.VectorSubcoreMesh(core_axis_name="c", subcore_axis_name="s"),
    scratch_types=[pltpu.VMEM((S, D), jnp.float32)]
    + _scr_idx + _scr_ge + _scr_go + _scr_sb + _scr_sem,
    compiler_params=pltpu.CompilerParams(use_tc_tiling_on_sc=False),
)(_emb_body)


def _mask_body(ids_ref, m_ref):
    m_ref[...] = ids_ref[...] == EOS


_mask = pl.pallas_call(
    _mask_body,
    out_shape=jax.ShapeDtypeStruct((B, S), jnp.bool_),
)


def kernel(input_ids, text_embedding, positional_embedding):
    ids = input_ids.astype(jnp.int32)
    x = _emb(ids, text_embedding, positional_embedding)
    tokens = x.reshape(B, S, D)
    mask = _mask(ids)
    return tokens, mask
